# Initial kernel scaffold; baseline (speedup 1.0000x reference)
import jax
import jax.numpy as jnp
from jax.experimental import pallas as pl


def kernel(x, conv1_w, gamma1, beta1, W_gat, att_src, att_dst, bias_gat, gamma_g, beta_g, conv3_w, gamma3, beta3, edge_index):
    raise NotImplementedError("write your pallas kernel here")


# speedup vs baseline: 1.3146x; 1.3146x over previous
import functools

import jax
import jax.numpy as jnp
from jax.experimental import pallas as pl
from jax.experimental.pallas import tpu as pltpu

_EPS = 1e-05
_NE = 22
_T = 1000
_K1 = 32
_NH = 4
_HD = 8
_F2 = 32
_NF = 40
_NEDGE = 110
_K3 = 16
_T8 = 125
_T4 = 31

_HI = jax.lax.Precision.HIGHEST


def _fwd_kernel(x_ref, g2_ref, b0_ref, ei_ref, sg_ref, bg_ref, w3_ref,
                s3_ref, b3_ref, p8_ref, p4_ref, o_ref):
    f32 = jnp.float32
    xp = x_ref[0]

    P = jnp.concatenate([xp[:, k:k + _T] for k in range(_K1)], axis=0)
    NF2 = jax.lax.dot_general(g2_ref[...], P, (((1,), (0,)), ((), ())),
                              preferred_element_type=f32, precision=_HI)
    NF = NF2.reshape(_NE, _NF, _T) + b0_ref[...][None]

    ei = ei_ref[...]
    src = ei[0:1, :_NEDGE]
    dst = ei[1:2, :_NEDGE]
    it = jax.lax.broadcasted_iota(jnp.int32, (_NE, _NEDGE), 0)
    eq_s = (it == src).astype(f32)
    eq_d = (it == dst).astype(f32)
    maskf = jax.lax.dot_general(eq_d, eq_s, (((1,), (1,)), ((), ())),
                                preferred_element_type=f32, precision=_HI)
    negf = (maskf - 1.0) * f32(1e30)

    gm_parts = []
    for h in range(_NH):
        ash = NF[:, 32 + h, :]
        adh = NF[:, 36 + h, :]
        E = ash[None, :, :] + adh[:, None, :]
        E = jnp.where(E > 0, E, 0.2 * E)
        E = E + negf[:, :, None]
        m = jnp.max(E, axis=1, keepdims=True)
        ee = jnp.exp(E - m)
        denom = jnp.sum(ee, axis=1, keepdims=True)
        pn = ee * (1.0 / (denom + 1e-16))
        wcol = jnp.sum(pn, axis=0)
        nfh = NF[:, _HD * h:_HD * h + _HD, :]
        gm_parts.append(jnp.sum(nfh * wcol[:, None, :], axis=0))
    gm = jnp.concatenate(gm_parts, axis=0)

    z = gm * sg_ref[...] + bg_ref[...]
    z = jnp.where(z > 0, z, jnp.exp(jnp.minimum(z, 0.0)) - 1.0)
    z8 = jax.lax.dot_general(z, p8_ref[...], (((1,), (0,)), ((), ())),
                             preferred_element_type=f32, precision=_HI)

    zpad = jnp.concatenate(
        [jnp.zeros((_F2, 7), f32), z8, jnp.zeros((_F2, 8), f32)], axis=1)
    zst = jnp.concatenate([zpad[:, k:k + _T8] for k in range(_K3)], axis=0)
    c3 = jax.lax.dot_general(w3_ref[...], zst, (((1,), (0,)), ((), ())),
                             preferred_element_type=f32, precision=_HI)
    c3 = c3 * s3_ref[...] + b3_ref[...]
    c3 = jnp.where(c3 > 0, c3, jnp.exp(jnp.minimum(c3, 0.0)) - 1.0)
    o = jax.lax.dot_general(c3, p4_ref[...], (((1,), (0,)), ((), ())),
                            preferred_element_type=f32, precision=_HI)
    o_ref[0] = o


@functools.partial(jax.jit, static_argnames=("interpret",))
def _run(x, conv1_w, gamma1, beta1, W_gat, att_src, att_dst, bias_gat,
         gamma_g, beta_g, conv3_w, gamma3, beta3, edge_index,
         interpret=False):
    f32 = jnp.float32
    B = x.shape[0]

    inv_sqrt = 1.0 / jnp.sqrt(1.0 + _EPS)
    scale1 = gamma1 * inv_sqrt
    w1s = conv1_w[:, 0, 0, :] * scale1[:, None]
    Wf = w1s.T @ W_gat
    b0 = beta1 @ W_gat
    eye4 = jnp.eye(_NH, dtype=f32)
    as_mat = (att_src[:, :, None] * eye4[:, None, :]).reshape(_F2, _NH)
    ad_mat = (att_dst[:, :, None] * eye4[:, None, :]).reshape(_F2, _NH)
    Wf_ext = jnp.concatenate([Wf, Wf @ as_mat, Wf @ ad_mat], axis=1)
    b0_ext = jnp.concatenate([b0, b0 @ as_mat, b0 @ ad_mat])
    b0c = b0_ext[:, None]

    eye22 = jnp.eye(_NE, dtype=f32)
    G2 = (Wf_ext.T[None, :, :, None] * eye22[:, None, None, :]
          ).reshape(_NE * _NF, _K1 * _NE)

    sg = gamma_g * inv_sqrt
    sg2 = (sg / f32(_NE))[:, None]
    bg2 = (bias_gat * sg + beta_g)[:, None]

    W3flat = jnp.transpose(conv3_w[:, :, 0, :], (0, 2, 1)).reshape(_F2, _K3 * _F2)
    s3 = (gamma3 * inv_sqrt)[:, None]
    b3 = beta3[:, None]

    p8m = ((jnp.arange(_T)[:, None] // 8) == jnp.arange(_T8)[None, :]
           ).astype(f32) / 8.0
    p4m = ((jnp.arange(_T8)[:, None] // 4) == jnp.arange(_T4)[None, :]
           ).astype(f32) / 4.0

    ei_pad = jnp.zeros((8, 128), jnp.int32).at[:2, :_NEDGE].set(edge_index)
    xpad = jnp.pad(x[:, 0], ((0, 0), (0, 0), (15, 17)))

    full = lambda a: pl.BlockSpec(a.shape, lambda b: (0,) * a.ndim)
    out = pl.pallas_call(
        _fwd_kernel,
        grid=(B,),
        in_specs=[
            pl.BlockSpec((1, _NE, _T + _K1), lambda b: (b, 0, 0)),
            full(G2), full(b0c), full(ei_pad), full(sg2), full(bg2),
            full(W3flat), full(s3), full(b3), full(p8m), full(p4m),
        ],
        out_specs=pl.BlockSpec((1, _F2, _T4), lambda b: (b, 0, 0)),
        out_shape=jax.ShapeDtypeStruct((B, _F2, _T4), f32),
        compiler_params=pltpu.CompilerParams(
            dimension_semantics=("parallel",)),
        interpret=interpret,
    )(xpad, G2, b0c, ei_pad, sg2, bg2, W3flat, s3, b3, p8m, p4m)
    return out[:, :, None, :]


def kernel(x, conv1_w, gamma1, beta1, W_gat, att_src, att_dst, bias_gat,
           gamma_g, beta_g, conv3_w, gamma3, beta3, edge_index):
    return _run(x, conv1_w, gamma1, beta1, W_gat, att_src, att_dst, bias_gat,
                gamma_g, beta_g, conv3_w, gamma3, beta3, edge_index)


# speedup vs baseline: 1.4470x; 1.1007x over previous
import functools

import jax
import jax.numpy as jnp
from jax.experimental import pallas as pl
from jax.experimental.pallas import tpu as pltpu

_EPS = 1e-05
_NE = 22
_T = 1000
_K1 = 32
_NH = 4
_HD = 8
_F2 = 32
_NF = 40
_NEDGE = 110
_K3 = 16
_T8 = 125
_T4 = 31

_HI = jax.lax.Precision.HIGHEST


def _fwd_kernel(x_ref, g2_ref, b0_ref, ei_ref, sg_ref, bg_ref, w3_ref,
                s3_ref, b3_ref, p8_ref, p4_ref, o_ref):
    f32 = jnp.float32
    xp = x_ref[0]

    P = jnp.concatenate([xp[:, k:k + _T] for k in range(_K1)], axis=0)
    NF2 = jax.lax.dot_general(g2_ref[...], P, (((1,), (0,)), ((), ())),
                              preferred_element_type=f32,
                              precision=jax.lax.Precision.DEFAULT)
    NF = NF2.reshape(_NE, _NF, _T) + b0_ref[...][None]

    ei = ei_ref[...]
    src = ei[0:1, :_NEDGE]
    dst = ei[1:2, :_NEDGE]
    it = jax.lax.broadcasted_iota(jnp.int32, (_NE, _NEDGE), 0)
    eq_s = (it == src).astype(f32)
    eq_d = (it == dst).astype(f32)
    maskf = jax.lax.dot_general(eq_d, eq_s, (((1,), (1,)), ((), ())),
                                preferred_element_type=f32, precision=_HI)
    negf = (maskf - 1.0) * f32(1e30)

    gm_parts = []
    for h in range(_NH):
        ash = NF[:, 32 + h, :]
        adh = NF[:, 36 + h, :]
        E = ash[None, :, :] + adh[:, None, :]
        E = jnp.where(E > 0, E, 0.2 * E)
        E = E + negf[:, :, None]
        m = jnp.max(E, axis=1, keepdims=True)
        ee = jnp.exp(E - m)
        denom = jnp.sum(ee, axis=1, keepdims=True)
        pn = ee * (1.0 / (denom + 1e-16))
        wcol = jnp.sum(pn, axis=0)
        nfh = NF[:, _HD * h:_HD * h + _HD, :]
        gm_parts.append(jnp.sum(nfh * wcol[:, None, :], axis=0))
    gm = jnp.concatenate(gm_parts, axis=0)

    z = gm * sg_ref[...] + bg_ref[...]
    z = jnp.where(z > 0, z, jnp.exp(jnp.minimum(z, 0.0)) - 1.0)
    z8 = jax.lax.dot_general(z, p8_ref[...], (((1,), (0,)), ((), ())),
                             preferred_element_type=f32, precision=_HI)

    zpad = jnp.concatenate(
        [jnp.zeros((_F2, 7), f32), z8, jnp.zeros((_F2, 8), f32)], axis=1)
    zst = jnp.concatenate([zpad[:, k:k + _T8] for k in range(_K3)], axis=0)
    c3 = jax.lax.dot_general(w3_ref[...], zst, (((1,), (0,)), ((), ())),
                             preferred_element_type=f32, precision=_HI)
    c3 = c3 * s3_ref[...] + b3_ref[...]
    c3 = jnp.where(c3 > 0, c3, jnp.exp(jnp.minimum(c3, 0.0)) - 1.0)
    o = jax.lax.dot_general(c3, p4_ref[...], (((1,), (0,)), ((), ())),
                            preferred_element_type=f32, precision=_HI)
    o_ref[0] = o


@functools.partial(jax.jit, static_argnames=("interpret",))
def _run(x, conv1_w, gamma1, beta1, W_gat, att_src, att_dst, bias_gat,
         gamma_g, beta_g, conv3_w, gamma3, beta3, edge_index,
         interpret=False):
    f32 = jnp.float32
    B = x.shape[0]

    inv_sqrt = 1.0 / jnp.sqrt(1.0 + _EPS)
    scale1 = gamma1 * inv_sqrt
    w1s = conv1_w[:, 0, 0, :] * scale1[:, None]
    Wf = w1s.T @ W_gat
    b0 = beta1 @ W_gat
    eye4 = jnp.eye(_NH, dtype=f32)
    as_mat = (att_src[:, :, None] * eye4[:, None, :]).reshape(_F2, _NH)
    ad_mat = (att_dst[:, :, None] * eye4[:, None, :]).reshape(_F2, _NH)
    Wf_ext = jnp.concatenate([Wf, Wf @ as_mat, Wf @ ad_mat], axis=1)
    b0_ext = jnp.concatenate([b0, b0 @ as_mat, b0 @ ad_mat])
    b0c = b0_ext[:, None]

    eye22 = jnp.eye(_NE, dtype=f32)
    G2 = (Wf_ext.T[None, :, :, None] * eye22[:, None, None, :]
          ).reshape(_NE * _NF, _K1 * _NE)

    sg = gamma_g * inv_sqrt
    sg2 = (sg / f32(_NE))[:, None]
    bg2 = (bias_gat * sg + beta_g)[:, None]

    W3flat = jnp.transpose(conv3_w[:, :, 0, :], (0, 2, 1)).reshape(_F2, _K3 * _F2)
    s3 = (gamma3 * inv_sqrt)[:, None]
    b3 = beta3[:, None]

    p8m = ((jnp.arange(_T)[:, None] // 8) == jnp.arange(_T8)[None, :]
           ).astype(f32) / 8.0
    p4m = ((jnp.arange(_T8)[:, None] // 4) == jnp.arange(_T4)[None, :]
           ).astype(f32) / 4.0

    ei_pad = jnp.zeros((8, 128), jnp.int32).at[:2, :_NEDGE].set(edge_index)
    xpad = jnp.pad(x[:, 0], ((0, 0), (0, 0), (15, 17)))

    full = lambda a: pl.BlockSpec(a.shape, lambda b: (0,) * a.ndim)
    out = pl.pallas_call(
        _fwd_kernel,
        grid=(B,),
        in_specs=[
            pl.BlockSpec((1, _NE, _T + _K1), lambda b: (b, 0, 0)),
            full(G2), full(b0c), full(ei_pad), full(sg2), full(bg2),
            full(W3flat), full(s3), full(b3), full(p8m), full(p4m),
        ],
        out_specs=pl.BlockSpec((1, _F2, _T4), lambda b: (b, 0, 0)),
        out_shape=jax.ShapeDtypeStruct((B, _F2, _T4), f32),
        compiler_params=pltpu.CompilerParams(
            dimension_semantics=("parallel",)),
        interpret=interpret,
    )(xpad, G2, b0c, ei_pad, sg2, bg2, W3flat, s3, b3, p8m, p4m)
    return out[:, :, None, :]


def kernel(x, conv1_w, gamma1, beta1, W_gat, att_src, att_dst, bias_gat,
           gamma_g, beta_g, conv3_w, gamma3, beta3, edge_index):
    return _run(x, conv1_w, gamma1, beta1, W_gat, att_src, att_dst, bias_gat,
                gamma_g, beta_g, conv3_w, gamma3, beta3, edge_index)


# speedup vs baseline: 14.3540x; 9.9201x over previous
import functools

import jax
import jax.numpy as jnp
from jax.experimental import pallas as pl
from jax.experimental.pallas import tpu as pltpu

_EPS = 1e-05
_NE = 22
_T = 1000
_K1 = 32
_NH = 4
_HD = 8
_F2 = 32
_NF = 40
_NEDGE = 110
_E4 = _NEDGE * _NH
_A4 = _NE * _NH
_K3 = 16
_T8 = 125
_T4 = 31

_HI = jax.lax.Precision.HIGHEST
_DF = jax.lax.Precision.DEFAULT


def _mm(a, b, precision=_DF):
    return jax.lax.dot_general(a, b, (((1,), (0,)), ((), ())),
                               preferred_element_type=jnp.float32,
                               precision=precision)


def _lrelu(v):
    return jnp.where(v > 0, v, 0.2 * v)


def _fwd_kernel(x_ref, g2_ref, b0_ref, ei_ref, eit_ref, rh_ref, q_ref,
                sg_ref, bg_ref, w3_ref, s3_ref, b3_ref, p8_ref, p4_ref,
                o_ref):
    f32 = jnp.float32
    i32 = jnp.int32
    xp = x_ref[0]

    P = jnp.concatenate([xp[:, k:k + _T] for k in range(_K1)], axis=0)
    NF2 = _mm(g2_ref[...], P) + b0_ref[...]
    nf_feat = NF2[:_F2 * _NE]
    ash_all = NF2[_F2 * _NE:(_F2 + _NH) * _NE]
    adh_all = NF2[(_F2 + _NH) * _NE:]

    ei = ei_ref[...]
    src_r = ei[0:1, :_NEDGE]
    dst_r = ei[1:2, :_NEDGE]
    eit = eit_ref[...]
    src_c = eit[:_NEDGE, 0:1]
    dst_c = eit[:_NEDGE, 1:2]
    src_c4 = jnp.concatenate([src_c] * _NH, axis=0)
    dst_c4 = jnp.concatenate([dst_c] * _NH, axis=0)
    src_r4 = jnp.concatenate([src_r] * _NH, axis=1)
    dst_r4 = jnp.concatenate([dst_r] * _NH, axis=1)

    hrow_g = jax.lax.broadcasted_iota(i32, (_E4, _A4), 0) // _NEDGE
    hcol_g = jax.lax.broadcasted_iota(i32, (_E4, _A4), 1) // _NE
    ncol_g = jax.lax.broadcasted_iota(i32, (_E4, _A4), 1) % _NE
    same_h_g = hrow_g == hcol_g
    Ssrc4 = (same_h_g & (ncol_g == src_c4)).astype(f32)
    Sdst4 = (same_h_g & (ncol_g == dst_c4)).astype(f32)

    hrow_s = jax.lax.broadcasted_iota(i32, (_A4, _E4), 0) // _NE
    nrow_s = jax.lax.broadcasted_iota(i32, (_A4, _E4), 0) % _NE
    hcol_s = jax.lax.broadcasted_iota(i32, (_A4, _E4), 1) // _NEDGE
    same_h_s = hrow_s == hcol_s
    Dsum4 = (same_h_s & (nrow_s == dst_r4)).astype(f32)
    Ssum4 = (same_h_s & (nrow_s == src_r4)).astype(f32)

    es = _mm(Ssrc4, ash_all)
    ed = _mm(Sdst4, adh_all)
    E = _lrelu(es + ed)

    c_parts = []
    for h in range(_NH):
        ash_h = ash_all[_NE * h:_NE * (h + 1)]
        amax_h = jnp.max(ash_h, axis=0, keepdims=True)
        adh_h = adh_all[_NE * h:_NE * (h + 1)]
        c_parts.append(_lrelu(jnp.broadcast_to(amax_h, (_NE, _T)) + adh_h))
    c_all = jnp.concatenate(c_parts, axis=0)

    ce = _mm(Sdst4, c_all)
    ee = jnp.exp(E - ce)
    denom = _mm(Dsum4, ee)
    inv = 1.0 / (denom + 1e-16)
    iedge = _mm(Sdst4, inv)
    pn = ee * iedge
    wcol_all = _mm(Ssum4, pn)

    wrep = _mm(rh_ref[...], wcol_all)
    gm = _mm(q_ref[...], nf_feat * wrep)

    z = gm * sg_ref[...] + bg_ref[...]
    z = jnp.where(z > 0, z, jnp.exp(jnp.minimum(z, 0.0)) - 1.0)
    z8 = _mm(z, p8_ref[...])

    zpad = jnp.concatenate(
        [jnp.zeros((_F2, 7), f32), z8, jnp.zeros((_F2, 8), f32)], axis=1)
    zst = jnp.concatenate([zpad[:, k:k + _T8] for k in range(_K3)], axis=0)
    c3 = _mm(w3_ref[...], zst)
    c3 = c3 * s3_ref[...] + b3_ref[...]
    c3 = jnp.where(c3 > 0, c3, jnp.exp(jnp.minimum(c3, 0.0)) - 1.0)
    o_ref[0] = _mm(c3, p4_ref[...])


@functools.partial(jax.jit, static_argnames=("interpret",))
def _run(x, conv1_w, gamma1, beta1, W_gat, att_src, att_dst, bias_gat,
         gamma_g, beta_g, conv3_w, gamma3, beta3, edge_index,
         interpret=False):
    f32 = jnp.float32
    B = x.shape[0]

    inv_sqrt = 1.0 / jnp.sqrt(1.0 + _EPS)
    scale1 = gamma1 * inv_sqrt
    w1s = conv1_w[:, 0, 0, :] * scale1[:, None]
    Wf = w1s.T @ W_gat
    b0 = beta1 @ W_gat
    eye4 = jnp.eye(_NH, dtype=f32)
    as_mat = (att_src[:, :, None] * eye4[:, None, :]).reshape(_F2, _NH)
    ad_mat = (att_dst[:, :, None] * eye4[:, None, :]).reshape(_F2, _NH)
    Wf_ext = jnp.concatenate([Wf, Wf @ as_mat, Wf @ ad_mat], axis=1)
    b0_ext = jnp.concatenate([b0, b0 @ as_mat, b0 @ ad_mat])
    b0col = jnp.repeat(b0_ext, _NE)[:, None]

    eye22 = jnp.eye(_NE, dtype=f32)
    G2 = (Wf_ext.T[:, None, :, None] * eye22[None, :, None, :]
          ).reshape(_NF * _NE, _K1 * _NE)

    jh = jnp.arange(_F2)
    hsel = ((jh[:, None] // _HD) == jnp.arange(_NH)[None, :]).astype(f32)
    Rh = (hsel[:, None, :, None] * eye22[None, :, None, :]
          ).reshape(_F2 * _NE, _NH * _NE)
    qsel = (jnp.arange(_F2)[:, None] == jnp.arange(_F2)[None, :]).astype(f32)
    Q = jnp.broadcast_to(qsel[:, :, None], (_F2, _F2, _NE)
                         ).reshape(_F2, _F2 * _NE)

    sg = gamma_g * inv_sqrt
    sg2 = (sg / f32(_NE))[:, None]
    bg2 = (bias_gat * sg + beta_g)[:, None]

    W3flat = jnp.transpose(conv3_w[:, :, 0, :], (0, 2, 1)).reshape(
        _F2, _K3 * _F2)
    s3 = (gamma3 * inv_sqrt)[:, None]
    b3 = beta3[:, None]

    p8m = ((jnp.arange(_T)[:, None] // 8) == jnp.arange(_T8)[None, :]
           ).astype(f32) / 8.0
    p4m = ((jnp.arange(_T8)[:, None] // 4) == jnp.arange(_T4)[None, :]
           ).astype(f32) / 4.0

    ei_pad = jnp.zeros((8, 128), jnp.int32).at[:2, :_NEDGE].set(edge_index)
    eit_pad = jnp.zeros((128, 8), jnp.int32).at[:_NEDGE, :2].set(
        edge_index.T)
    xpad = jnp.pad(x[:, 0], ((0, 0), (0, 0), (15, 17)))

    full = lambda a: pl.BlockSpec(a.shape, lambda b: (0,) * a.ndim)
    out = pl.pallas_call(
        _fwd_kernel,
        grid=(B,),
        in_specs=[
            pl.BlockSpec((1, _NE, _T + _K1), lambda b: (b, 0, 0)),
            full(G2), full(b0col), full(ei_pad), full(eit_pad),
            full(Rh), full(Q), full(sg2), full(bg2),
            full(W3flat), full(s3), full(b3), full(p8m), full(p4m),
        ],
        out_specs=pl.BlockSpec((1, _F2, _T4), lambda b: (b, 0, 0)),
        out_shape=jax.ShapeDtypeStruct((B, _F2, _T4), f32),
        compiler_params=pltpu.CompilerParams(
            dimension_semantics=("parallel",)),
        interpret=interpret,
    )(xpad, G2, b0col, ei_pad, eit_pad, Rh, Q, sg2, bg2, W3flat, s3, b3,
      p8m, p4m)
    return out[:, :, None, :]


def kernel(x, conv1_w, gamma1, beta1, W_gat, att_src, att_dst, bias_gat,
           gamma_g, beta_g, conv3_w, gamma3, beta3, edge_index):
    return _run(x, conv1_w, gamma1, beta1, W_gat, att_src, att_dst, bias_gat,
                gamma_g, beta_g, conv3_w, gamma3, beta3, edge_index)


# speedup vs baseline: 15.0593x; 1.0491x over previous
import functools

import jax
import jax.numpy as jnp
from jax.experimental import pallas as pl
from jax.experimental.pallas import tpu as pltpu

_EPS = 1e-05
_NE = 22
_T = 1000
_K1 = 32
_NH = 4
_HD = 8
_F2 = 32
_NF = 40
_NEDGE = 110
_E4 = _NEDGE * _NH
_A4 = _NE * _NH
_K3 = 16
_T8 = 125
_T4 = 31

_HI = jax.lax.Precision.HIGHEST
_DF = jax.lax.Precision.DEFAULT


def _mm(a, b, precision=_DF):
    return jax.lax.dot_general(a, b, (((1,), (0,)), ((), ())),
                               preferred_element_type=jnp.float32,
                               precision=precision)


def _lrelu(v):
    return jnp.where(v > 0, v, 0.2 * v)


def _fwd_kernel(x_ref, g2_ref, b0_ref, ei_ref, eit_ref, rh_ref, q_ref,
                sg_ref, bg_ref, w3_ref, s3_ref, b3_ref, p8_ref, p4_ref,
                o_ref):
    f32 = jnp.float32
    i32 = jnp.int32
    xp = x_ref[0]

    P = jnp.concatenate([xp[:, k:k + _T] for k in range(_K1)], axis=0)
    NF2 = _mm(g2_ref[...], P) + b0_ref[...]
    nf_feat = NF2[:_F2 * _NE]
    ash_all = NF2[_F2 * _NE:(_F2 + _NH) * _NE]
    adh_all = NF2[(_F2 + _NH) * _NE:]

    ei = ei_ref[...]
    src_r = ei[0:1, :_NEDGE]
    dst_r = ei[1:2, :_NEDGE]
    eit = eit_ref[...]
    src_c = eit[:_NEDGE, 0:1]
    dst_c = eit[:_NEDGE, 1:2]
    src_c4 = jnp.concatenate([src_c] * _NH, axis=0)
    dst_c4 = jnp.concatenate([dst_c] * _NH, axis=0)
    src_r4 = jnp.concatenate([src_r] * _NH, axis=1)
    dst_r4 = jnp.concatenate([dst_r] * _NH, axis=1)

    hrow_g = jax.lax.broadcasted_iota(i32, (_E4, _A4), 0) // _NEDGE
    hcol_g = jax.lax.broadcasted_iota(i32, (_E4, _A4), 1) // _NE
    ncol_g = jax.lax.broadcasted_iota(i32, (_E4, _A4), 1) % _NE
    same_h_g = hrow_g == hcol_g
    Ssrc4 = (same_h_g & (ncol_g == src_c4)).astype(f32)
    Sdst4 = (same_h_g & (ncol_g == dst_c4)).astype(f32)

    hrow_s = jax.lax.broadcasted_iota(i32, (_A4, _E4), 0) // _NE
    nrow_s = jax.lax.broadcasted_iota(i32, (_A4, _E4), 0) % _NE
    hcol_s = jax.lax.broadcasted_iota(i32, (_A4, _E4), 1) // _NEDGE
    same_h_s = hrow_s == hcol_s
    Dsum4 = (same_h_s & (nrow_s == dst_r4)).astype(f32)
    Ssum4 = (same_h_s & (nrow_s == src_r4)).astype(f32)

    SS = jnp.concatenate([Ssrc4, Sdst4], axis=1)
    E = _lrelu(_mm(SS, NF2[_F2 * _NE:]))

    c_parts = []
    for h in range(_NH):
        amax_h = jnp.max(ash_all[_NE * h:_NE * (h + 1)], axis=0,
                         keepdims=True)
        dmax_h = jnp.max(adh_all[_NE * h:_NE * (h + 1)], axis=0,
                         keepdims=True)
        c_parts.append(jnp.broadcast_to(_lrelu(amax_h + dmax_h),
                                        (_NEDGE, _T)))
    c440 = jnp.concatenate(c_parts, axis=0)
    ee = jnp.exp(E - c440)
    denom = _mm(Dsum4, ee)
    inv = 1.0 / (denom + 1e-16)
    iedge = _mm(Sdst4, inv)
    pn = ee * iedge
    wcol_all = _mm(Ssum4, pn)

    wrep = _mm(rh_ref[...], wcol_all)
    gm = _mm(q_ref[...], nf_feat * wrep)

    z = gm * sg_ref[...] + bg_ref[...]
    z = jnp.where(z > 0, z, jnp.exp(jnp.minimum(z, 0.0)) - 1.0)
    z8 = _mm(z, p8_ref[...])

    zpad = jnp.concatenate(
        [jnp.zeros((_F2, 7), f32), z8, jnp.zeros((_F2, 8), f32)], axis=1)
    zst = jnp.concatenate([zpad[:, k:k + _T8] for k in range(_K3)], axis=0)
    c3 = _mm(w3_ref[...], zst)
    c3 = c3 * s3_ref[...] + b3_ref[...]
    c3 = jnp.where(c3 > 0, c3, jnp.exp(jnp.minimum(c3, 0.0)) - 1.0)
    o_ref[0] = _mm(c3, p4_ref[...])


@functools.partial(jax.jit, static_argnames=("interpret",))
def _run(x, conv1_w, gamma1, beta1, W_gat, att_src, att_dst, bias_gat,
         gamma_g, beta_g, conv3_w, gamma3, beta3, edge_index,
         interpret=False):
    f32 = jnp.float32
    B = x.shape[0]

    inv_sqrt = 1.0 / jnp.sqrt(1.0 + _EPS)
    scale1 = gamma1 * inv_sqrt
    w1s = conv1_w[:, 0, 0, :] * scale1[:, None]
    Wf = w1s.T @ W_gat
    b0 = beta1 @ W_gat
    eye4 = jnp.eye(_NH, dtype=f32)
    as_mat = (att_src[:, :, None] * eye4[:, None, :]).reshape(_F2, _NH)
    ad_mat = (att_dst[:, :, None] * eye4[:, None, :]).reshape(_F2, _NH)
    Wf_ext = jnp.concatenate([Wf, Wf @ as_mat, Wf @ ad_mat], axis=1)
    b0_ext = jnp.concatenate([b0, b0 @ as_mat, b0 @ ad_mat])
    b0col = jnp.repeat(b0_ext, _NE)[:, None]

    eye22 = jnp.eye(_NE, dtype=f32)
    G2 = (Wf_ext.T[:, None, :, None] * eye22[None, :, None, :]
          ).reshape(_NF * _NE, _K1 * _NE)

    jh = jnp.arange(_F2)
    hsel = ((jh[:, None] // _HD) == jnp.arange(_NH)[None, :]).astype(f32)
    Rh = (hsel[:, None, :, None] * eye22[None, :, None, :]
          ).reshape(_F2 * _NE, _NH * _NE)
    qsel = (jnp.arange(_F2)[:, None] == jnp.arange(_F2)[None, :]).astype(f32)
    Q = jnp.broadcast_to(qsel[:, :, None], (_F2, _F2, _NE)
                         ).reshape(_F2, _F2 * _NE)

    sg = gamma_g * inv_sqrt
    sg2 = (sg / f32(_NE))[:, None]
    bg2 = (bias_gat * sg + beta_g)[:, None]

    W3flat = jnp.transpose(conv3_w[:, :, 0, :], (0, 2, 1)).reshape(
        _F2, _K3 * _F2)
    s3 = (gamma3 * inv_sqrt)[:, None]
    b3 = beta3[:, None]

    p8m = ((jnp.arange(_T)[:, None] // 8) == jnp.arange(_T8)[None, :]
           ).astype(f32) / 8.0
    p4m = ((jnp.arange(_T8)[:, None] // 4) == jnp.arange(_T4)[None, :]
           ).astype(f32) / 4.0

    ei_pad = jnp.zeros((8, 128), jnp.int32).at[:2, :_NEDGE].set(edge_index)
    eit_pad = jnp.zeros((128, 8), jnp.int32).at[:_NEDGE, :2].set(
        edge_index.T)
    xpad = jnp.pad(x[:, 0], ((0, 0), (0, 0), (15, 17)))

    full = lambda a: pl.BlockSpec(a.shape, lambda b: (0,) * a.ndim)
    out = pl.pallas_call(
        _fwd_kernel,
        grid=(B,),
        in_specs=[
            pl.BlockSpec((1, _NE, _T + _K1), lambda b: (b, 0, 0)),
            full(G2), full(b0col), full(ei_pad), full(eit_pad),
            full(Rh), full(Q), full(sg2), full(bg2),
            full(W3flat), full(s3), full(b3), full(p8m), full(p4m),
        ],
        out_specs=pl.BlockSpec((1, _F2, _T4), lambda b: (b, 0, 0)),
        out_shape=jax.ShapeDtypeStruct((B, _F2, _T4), f32),
        compiler_params=pltpu.CompilerParams(
            dimension_semantics=("parallel",)),
        interpret=interpret,
    )(xpad, G2, b0col, ei_pad, eit_pad, Rh, Q, sg2, bg2, W3flat, s3, b3,
      p8m, p4m)
    return out[:, :, None, :]


def kernel(x, conv1_w, gamma1, beta1, W_gat, att_src, att_dst, bias_gat,
           gamma_g, beta_g, conv3_w, gamma3, beta3, edge_index):
    return _run(x, conv1_w, gamma1, beta1, W_gat, att_src, att_dst, bias_gat,
                gamma_g, beta_g, conv3_w, gamma3, beta3, edge_index)


# speedup vs baseline: 15.6921x; 1.0420x over previous
import functools

import jax
import jax.numpy as jnp
from jax.experimental import pallas as pl
from jax.experimental.pallas import tpu as pltpu

_EPS = 1e-05
_NE = 22
_T = 1000
_K1 = 32
_NH = 4
_HD = 8
_F2 = 32
_NF = 40
_NEDGE = 110
_E4 = _NEDGE * _NH
_A4 = _NE * _NH
_K3 = 16
_T8 = 125
_T4 = 31
_NB = 2
_TW = _NB * _T
_XW = 1032

_HI = jax.lax.Precision.HIGHEST
_DF = jax.lax.Precision.DEFAULT


def _mm(a, b, precision=_DF):
    return jax.lax.dot_general(a, b, (((1,), (0,)), ((), ())),
                               preferred_element_type=jnp.float32,
                               precision=precision)


def _lrelu(v):
    return jnp.where(v > 0, v, 0.2 * v)


def _fwd_kernel(x_ref, g2_ref, b0_ref, ei_ref, eit_ref, rh_ref, q_ref,
                sg_ref, bg_ref, w3_ref, s3_ref, b3_ref, p8_ref, p4_ref,
                o_ref):
    f32 = jnp.float32
    i32 = jnp.int32
    xp = x_ref[0]

    P = jnp.concatenate(
        [jnp.concatenate([xp[:, _XW * b + k:_XW * b + k + _T]
                          for b in range(_NB)], axis=1)
         for k in range(_K1)], axis=0)
    NF2 = _mm(g2_ref[...], P) + b0_ref[...]
    nf_feat = NF2[:_F2 * _NE]
    ash_all = NF2[_F2 * _NE:(_F2 + _NH) * _NE]
    adh_all = NF2[(_F2 + _NH) * _NE:]

    ei = ei_ref[...]
    src_r = ei[0:1, :_NEDGE]
    dst_r = ei[1:2, :_NEDGE]
    eit = eit_ref[...]
    src_c = eit[:_NEDGE, 0:1]
    dst_c = eit[:_NEDGE, 1:2]
    src_c4 = jnp.concatenate([src_c] * _NH, axis=0)
    dst_c4 = jnp.concatenate([dst_c] * _NH, axis=0)
    src_r4 = jnp.concatenate([src_r] * _NH, axis=1)
    dst_r4 = jnp.concatenate([dst_r] * _NH, axis=1)

    hrow_g = jax.lax.broadcasted_iota(i32, (_E4, _A4), 0) // _NEDGE
    hcol_g = jax.lax.broadcasted_iota(i32, (_E4, _A4), 1) // _NE
    ncol_g = jax.lax.broadcasted_iota(i32, (_E4, _A4), 1) % _NE
    same_h_g = hrow_g == hcol_g
    Ssrc4 = (same_h_g & (ncol_g == src_c4)).astype(f32)
    Sdst4 = (same_h_g & (ncol_g == dst_c4)).astype(f32)

    hrow_s = jax.lax.broadcasted_iota(i32, (_A4, _E4), 0) // _NE
    nrow_s = jax.lax.broadcasted_iota(i32, (_A4, _E4), 0) % _NE
    hcol_s = jax.lax.broadcasted_iota(i32, (_A4, _E4), 1) // _NEDGE
    same_h_s = hrow_s == hcol_s
    Dsum4 = (same_h_s & (nrow_s == dst_r4)).astype(f32)
    Ssum4 = (same_h_s & (nrow_s == src_r4)).astype(f32)

    SS = jnp.concatenate([Ssrc4, Sdst4], axis=1)
    E = _lrelu(_mm(SS, NF2[_F2 * _NE:]))

    c_parts = []
    for h in range(_NH):
        amax_h = jnp.max(ash_all[_NE * h:_NE * (h + 1)], axis=0,
                         keepdims=True)
        dmax_h = jnp.max(adh_all[_NE * h:_NE * (h + 1)], axis=0,
                         keepdims=True)
        c_parts.append(jnp.broadcast_to(_lrelu(amax_h + dmax_h),
                                        (_NEDGE, _TW)))
    c440 = jnp.concatenate(c_parts, axis=0)
    ee = jnp.exp(E - c440)
    denom = _mm(Dsum4, ee)
    inv = 1.0 / (denom + 1e-16)
    iedge = _mm(Sdst4, inv)
    pn = ee * iedge
    wcol_all = _mm(Ssum4, pn)

    wrep = _mm(rh_ref[...], wcol_all)
    gm = _mm(q_ref[...], nf_feat * wrep)

    z = gm * sg_ref[...] + bg_ref[...]
    z = jnp.where(z > 0, z, jnp.exp(jnp.minimum(z, 0.0)) - 1.0)
    z8 = _mm(z, p8_ref[...])

    zpad = jnp.concatenate(
        [jnp.zeros((_F2, 7), f32), z8[:, :_T8], jnp.zeros((_F2, 15), f32),
         z8[:, _T8:], jnp.zeros((_F2, 8), f32)], axis=1)
    zst = jnp.concatenate([zpad[:, k:k + 265] for k in range(_K3)], axis=0)
    c3 = _mm(w3_ref[...], zst)
    c3 = c3 * s3_ref[...] + b3_ref[...]
    c3 = jnp.where(c3 > 0, c3, jnp.exp(jnp.minimum(c3, 0.0)) - 1.0)
    o2 = _mm(c3, p4_ref[...])
    o_ref[0] = o2[:, :_T4]
    o_ref[1] = o2[:, _T4:]


@functools.partial(jax.jit, static_argnames=("interpret",))
def _run(x, conv1_w, gamma1, beta1, W_gat, att_src, att_dst, bias_gat,
         gamma_g, beta_g, conv3_w, gamma3, beta3, edge_index,
         interpret=False):
    f32 = jnp.float32
    B = x.shape[0]

    inv_sqrt = 1.0 / jnp.sqrt(1.0 + _EPS)
    scale1 = gamma1 * inv_sqrt
    w1s = conv1_w[:, 0, 0, :] * scale1[:, None]
    Wf = w1s.T @ W_gat
    b0 = beta1 @ W_gat
    eye4 = jnp.eye(_NH, dtype=f32)
    as_mat = (att_src[:, :, None] * eye4[:, None, :]).reshape(_F2, _NH)
    ad_mat = (att_dst[:, :, None] * eye4[:, None, :]).reshape(_F2, _NH)
    Wf_ext = jnp.concatenate([Wf, Wf @ as_mat, Wf @ ad_mat], axis=1)
    b0_ext = jnp.concatenate([b0, b0 @ as_mat, b0 @ ad_mat])
    b0col = jnp.repeat(b0_ext, _NE)[:, None]

    eye22 = jnp.eye(_NE, dtype=f32)
    G2 = (Wf_ext.T[:, None, :, None] * eye22[None, :, None, :]
          ).reshape(_NF * _NE, _K1 * _NE)

    jh = jnp.arange(_F2)
    hsel = ((jh[:, None] // _HD) == jnp.arange(_NH)[None, :]).astype(f32)
    Rh = (hsel[:, None, :, None] * eye22[None, :, None, :]
          ).reshape(_F2 * _NE, _NH * _NE)
    qsel = (jnp.arange(_F2)[:, None] == jnp.arange(_F2)[None, :]).astype(f32)
    Q = jnp.broadcast_to(qsel[:, :, None], (_F2, _F2, _NE)
                         ).reshape(_F2, _F2 * _NE)

    sg = gamma_g * inv_sqrt
    sg2 = (sg / f32(_NE))[:, None]
    bg2 = (bias_gat * sg + beta_g)[:, None]

    W3flat = jnp.transpose(conv3_w[:, :, 0, :], (0, 2, 1)).reshape(
        _F2, _K3 * _F2)
    s3 = (gamma3 * inv_sqrt)[:, None]
    b3 = beta3[:, None]

    p8m = ((jnp.arange(_TW)[:, None] // 8) == jnp.arange(_NB * _T8)[None, :]
           ).astype(f32) / 8.0
    tp = jnp.arange(265)
    tt = tp % 140
    cglob = (tp // 140) * _T4 + tt // 4
    p4m = ((cglob[:, None] == jnp.arange(_NB * _T4)[None, :])
           & (tt < 124)[:, None]).astype(f32) / 4.0

    ei_pad = jnp.zeros((8, 128), jnp.int32).at[:2, :_NEDGE].set(edge_index)
    eit_pad = jnp.zeros((128, 8), jnp.int32).at[:_NEDGE, :2].set(
        edge_index.T)
    xpad = jnp.pad(x[:, 0], ((0, 0), (0, 0), (15, 17)))
    xg = xpad.reshape(B // _NB, _NB, _NE, _XW).transpose(0, 2, 1, 3
        ).reshape(B // _NB, _NE, _NB * _XW)

    full = lambda a: pl.BlockSpec(a.shape, lambda b: (0,) * a.ndim)
    out = pl.pallas_call(
        _fwd_kernel,
        grid=(B // _NB,),
        in_specs=[
            pl.BlockSpec((1, _NE, _NB * _XW), lambda b: (b, 0, 0)),
            full(G2), full(b0col), full(ei_pad), full(eit_pad),
            full(Rh), full(Q), full(sg2), full(bg2),
            full(W3flat), full(s3), full(b3), full(p8m), full(p4m),
        ],
        out_specs=pl.BlockSpec((_NB, _F2, _T4), lambda b: (b, 0, 0)),
        out_shape=jax.ShapeDtypeStruct((B, _F2, _T4), f32),
        compiler_params=pltpu.CompilerParams(
            dimension_semantics=("parallel",)),
        interpret=interpret,
    )(xg, G2, b0col, ei_pad, eit_pad, Rh, Q, sg2, bg2, W3flat, s3, b3,
      p8m, p4m)
    return out[:, :, None, :]


def kernel(x, conv1_w, gamma1, beta1, W_gat, att_src, att_dst, bias_gat,
           gamma_g, beta_g, conv3_w, gamma3, beta3, edge_index):
    return _run(x, conv1_w, gamma1, beta1, W_gat, att_src, att_dst, bias_gat,
                gamma_g, beta_g, conv3_w, gamma3, beta3, edge_index)
